# baseline (speedup 1.0000x reference)
import functools

import jax
import jax.numpy as jnp
from jax import lax
from jax.experimental import pallas as pl
from jax.experimental.pallas import tpu as pltpu
from jax.experimental.pallas import tpu_sc as plsc

B = 16384
D = 128
K = 16
GAMMA = 12.0
BETA = 0.5
G1 = 1.0
G2 = 1.0
EPS = 1e-12

NCORE = 2
NSUB = 16
NW = NCORE * NSUB
SPT = B // NW
GRP = 8
CHUNKS = SPT // GRP
L = 16
CPD = D // L


def _vsqrt(x):
    i = plsc.bitcast(x, jnp.int32)
    g = plsc.bitcast((i >> 1) + jnp.int32(0x1FBD1DF5), jnp.float32)
    for _ in range(3):
        g = 0.5 * (g + x / g)
    return g


def _sc_body(s0_hbm, s1_hbm, s2_hbm, neg_hbm, e_hbm, r_hbm, t_hbm, c_hbm,
             p_hbm, h_hbm, pa_hbm, out_hbm, part_hbm,
             s0m, s1m, s2m, negm, hcid, tcid, hpid, tpid,
             hbuf, tbuf, rbuf, htbuf, ttbuf, hcbuf, tcbuf, hpbuf, tpbuf,
             nebuf, score_v, part_v,
             sem_id0, sem_id1, sem_ne0, sem_ne1, sem_row0, sem_row1):
    wid = lax.axis_index("s") * NCORE + lax.axis_index("c")
    pltpu.sync_copy(s0_hbm.at[wid], s0m)
    pltpu.sync_copy(s1_hbm.at[wid], s1m)
    pltpu.sync_copy(s2_hbm.at[wid], s2m)
    pltpu.sync_copy(neg_hbm.at[wid], negm)

    id_sems = (sem_id0, sem_id1)
    ne_sems = (sem_ne0, sem_ne1)
    row_sems = (sem_row0, sem_row1)

    lane = lax.iota(jnp.int32, 16)
    wv = jnp.where(lane < 2, jnp.float32(G1),
                   jnp.where(lane < 4, jnp.float32(BETA),
                             jnp.where(lane < 6, jnp.float32(G2),
                                       jnp.where(lane == 6, jnp.float32(1.0),
                                                 jnp.float32(0.0)))))

    def issue_ids(j, p):
        i0 = s0m.at[pl.ds(j * GRP, GRP)]
        i2 = s2m.at[pl.ds(j * GRP, GRP)]
        pltpu.async_copy(h_hbm.at[i0], hcid.at[p], id_sems[p])
        pltpu.async_copy(h_hbm.at[i2], tcid.at[p], id_sems[p])
        pltpu.async_copy(pa_hbm.at[i0], hpid.at[p], id_sems[p])
        pltpu.async_copy(pa_hbm.at[i2], tpid.at[p], id_sems[p])

    def wait_ids(p):
        i0 = s0m.at[pl.ds(0, GRP)]
        for dst in (hcid, tcid, hpid, tpid):
            pltpu.make_async_copy(h_hbm.at[i0], dst.at[p], id_sems[p]).wait()

    def issue_ne(j, p):
        ineg = negm.at[pl.ds(j * GRP * K, GRP * K)]
        pltpu.async_copy(e_hbm.at[ineg], nebuf.at[p], ne_sems[p])

    def wait_ne(p):
        ineg = negm.at[pl.ds(0, GRP * K)]
        pltpu.make_async_copy(e_hbm.at[ineg], nebuf.at[p], ne_sems[p]).wait()

    def issue_rows(j, p):
        i0 = s0m.at[pl.ds(j * GRP, GRP)]
        i1 = s1m.at[pl.ds(j * GRP, GRP)]
        i2 = s2m.at[pl.ds(j * GRP, GRP)]
        ineg = negm.at[pl.ds(j * GRP * K, GRP * K)]
        sem = row_sems[p]
        pltpu.async_copy(e_hbm.at[i0], hbuf.at[p], sem)
        pltpu.async_copy(e_hbm.at[i2], tbuf.at[p], sem)
        pltpu.async_copy(r_hbm.at[i1], rbuf.at[p], sem)
        pltpu.async_copy(t_hbm.at[i0], htbuf.at[p], sem)
        pltpu.async_copy(t_hbm.at[i2], ttbuf.at[p], sem)
        pltpu.async_copy(t_hbm.at[ineg], nebuf.at[p], sem, add=True)
        pltpu.async_copy(c_hbm.at[hcid.at[p]], hcbuf.at[p], sem)
        pltpu.async_copy(c_hbm.at[tcid.at[p]], tcbuf.at[p], sem)
        pltpu.async_copy(p_hbm.at[hpid.at[p]], hpbuf.at[p], sem)
        pltpu.async_copy(p_hbm.at[tpid.at[p]], tpbuf.at[p], sem)

    def wait_rows(p):
        i0 = s0m.at[pl.ds(0, GRP)]
        ineg = negm.at[pl.ds(0, GRP * K)]
        sem = row_sems[p]
        for dst in (hbuf, tbuf, rbuf, htbuf, ttbuf, hcbuf, tcbuf, hpbuf,
                    tpbuf):
            pltpu.make_async_copy(e_hbm.at[i0], dst.at[p], sem).wait()
        pltpu.make_async_copy(e_hbm.at[ineg], nebuf.at[p], sem).wait()

    def compute(g, p, negacc_in):
        def sample_body(i, carry):
            scorevec, negacc = carry
            z = jnp.zeros((L,), jnp.float32)
            acc_l1 = z
            acc_ht = z
            acc_tt = z
            acc_hc = z
            acc_tc = z
            acc_hp = z
            acc_tp = z
            acc_tr = z
            preds = []
            for c in range(CPD):
                sl = pl.ds(c * L, L)
                h = hbuf[p, i, sl]
                ht = htbuf[p, i, sl]
                t = tbuf[p, i, sl]
                tt = ttbuf[p, i, sl]
                r = rbuf[p, i, sl]
                hc = hcbuf[p, i, sl]
                tc = tcbuf[p, i, sl]
                hp = hpbuf[p, i, sl]
                tp = tpbuf[p, i, sl]
                hcmb = 0.5 * (h + ht)
                tcmb = 0.5 * (t + tt)
                pred = hcmb + r
                u = pred - tcmb
                acc_l1 = acc_l1 + jnp.abs(u)
                dht = h - ht
                acc_ht = acc_ht + dht * dht
                dtt = t - tt
                acc_tt = acc_tt + dtt * dtt
                dhc = hcmb - hc
                acc_hc = acc_hc + dhc * dhc
                dtc = tcmb - tc
                acc_tc = acc_tc + dtc * dtc
                dhp = hc - hp
                acc_hp = acc_hp + dhp * dhp
                dtp = tc - tp
                acc_tp = acc_tp + dtp * dtp
                dtr = u - tt
                acc_tr = acc_tr + dtr * dtr
                preds.append(pred)

            qn = z
            for k in range(K):
                nacc = z
                for c in range(CPD):
                    sl = pl.ds(c * L, L)
                    ne = nebuf[p, i * K + k, sl]
                    dn = preds[c] - ne
                    nacc = nacc + dn * dn
                qn = jnp.where(lane == k, jnp.sum(nacc), qn)

            qd = jnp.full((L,), 1.0, jnp.float32)
            qd = jnp.where(lane == 0, 0.25 * jnp.sum(acc_ht), qd)
            qd = jnp.where(lane == 1, 0.25 * jnp.sum(acc_tt), qd)
            qd = jnp.where(lane == 2, jnp.sum(acc_hc), qd)
            qd = jnp.where(lane == 3, jnp.sum(acc_tc), qd)
            qd = jnp.where(lane == 4, jnp.sum(acc_hp), qd)
            qd = jnp.where(lane == 5, jnp.sum(acc_tp), qd)
            qd = jnp.where(lane == 6, jnp.sum(acc_tr), qd)
            sd = _vsqrt(qd + jnp.float32(EPS))
            sn = _vsqrt(qn + jnp.float32(EPS))
            penal = jnp.sum(wv * sd)
            score = jnp.float32(GAMMA) - jnp.sum(acc_l1) - penal
            return (jnp.where(lane == i, score, scorevec), negacc + sn)

        scorevec, negacc = lax.fori_loop(
            0, GRP, sample_body,
            (jnp.zeros((L,), jnp.float32), negacc_in))
        score_v[...] = scorevec
        pltpu.sync_copy(score_v.at[pl.ds(0, GRP)],
                        out_hbm.at[wid, pl.ds(g * GRP, GRP)])
        return negacc

    issue_ids(0, 0)
    issue_ids(1, 1)
    issue_ne(0, 0)
    issue_ne(1, 1)
    wait_ids(0)
    wait_ne(0)
    issue_rows(0, 0)

    def pair_body(tt, negacc):
        g0 = 2 * tt
        more = tt < CHUNKS // 2 - 1

        wait_ids(1)
        wait_ne(1)
        issue_rows(g0 + 1, 1)
        wait_rows(0)

        @pl.when(more)
        def _():
            issue_ids(g0 + 2, 0)

        negacc = compute(g0, 0, negacc)

        @pl.when(more)
        def _():
            issue_ne(g0 + 2, 0)
            wait_ids(0)
            wait_ne(0)
            issue_rows(g0 + 2, 0)

        wait_rows(1)

        @pl.when(more)
        def _():
            issue_ids(g0 + 3, 1)

        negacc = compute(g0 + 1, 1, negacc)

        @pl.when(more)
        def _():
            issue_ne(g0 + 3, 1)

        return negacc

    negacc = lax.fori_loop(0, CHUNKS // 2, pair_body,
                           jnp.zeros((L,), jnp.float32))

    for c in range(CPD):
        part_v[pl.ds(c * L, L)] = jnp.zeros((L,), jnp.float32)
    part_v[pl.ds(0, L)] = negacc
    pltpu.sync_copy(part_v, part_hbm.at[wid])


_sc_kernel = functools.partial(
    pl.kernel,
    out_type=(jax.ShapeDtypeStruct((NW, SPT), jnp.float32),
              jax.ShapeDtypeStruct((NW, D), jnp.float32)),
    mesh=plsc.VectorSubcoreMesh(core_axis_name="c", subcore_axis_name="s"),
    compiler_params=pltpu.CompilerParams(needs_layout_passes=False),
    scratch_types=[
        pltpu.VMEM((SPT,), jnp.int32),
        pltpu.VMEM((SPT,), jnp.int32),
        pltpu.VMEM((SPT,), jnp.int32),
        pltpu.VMEM((SPT * K,), jnp.int32),
        pltpu.VMEM((2, GRP), jnp.int32),
        pltpu.VMEM((2, GRP), jnp.int32),
        pltpu.VMEM((2, GRP), jnp.int32),
        pltpu.VMEM((2, GRP), jnp.int32),
        pltpu.VMEM((2, GRP, D), jnp.float32),
        pltpu.VMEM((2, GRP, D), jnp.float32),
        pltpu.VMEM((2, GRP, D), jnp.float32),
        pltpu.VMEM((2, GRP, D), jnp.float32),
        pltpu.VMEM((2, GRP, D), jnp.float32),
        pltpu.VMEM((2, GRP, D), jnp.float32),
        pltpu.VMEM((2, GRP, D), jnp.float32),
        pltpu.VMEM((2, GRP, D), jnp.float32),
        pltpu.VMEM((2, GRP, D), jnp.float32),
        pltpu.VMEM((2, GRP * K, D), jnp.float32),
        pltpu.VMEM((L,), jnp.float32),
        pltpu.VMEM((D,), jnp.float32),
        pltpu.SemaphoreType.DMA,
        pltpu.SemaphoreType.DMA,
        pltpu.SemaphoreType.DMA,
        pltpu.SemaphoreType.DMA,
        pltpu.SemaphoreType.DMA,
        pltpu.SemaphoreType.DMA,
    ],
)


def _run_sc(*args):
    return _sc_kernel(_sc_body)(*args)


def _combine_body(base_ref, part_ref, out_ref):
    negmean = jnp.sum(part_ref[...]) * jnp.float32(1.0 / (B * K))
    out_ref[...] = base_ref[...] + negmean


_combine = pl.pallas_call(
    _combine_body,
    out_shape=jax.ShapeDtypeStruct((NW, SPT), jnp.float32),
)


def kernel(sample, entity_embedding, relation_embedding, entity_text_embeddings,
           cluster_embedding, parent_cluster_embedding, entity_hierarchy,
           entity_parent, neg_idx):
    s0 = sample[:, 0].reshape(NW, SPT)
    s1 = sample[:, 1].reshape(NW, SPT)
    s2 = sample[:, 2].reshape(NW, SPT)
    neg2 = neg_idx.reshape(NW, SPT * K)
    base, parts = _run_sc(s0, s1, s2, neg2, entity_embedding,
                          relation_embedding, entity_text_embeddings,
                          cluster_embedding, parent_cluster_embedding,
                          entity_hierarchy, entity_parent)
    out = _combine(base, parts)
    return out.reshape(B, 1)


# speedup vs baseline: 1.9905x; 1.9905x over previous
import functools

import jax
import jax.numpy as jnp
from jax import lax
from jax.experimental import pallas as pl
from jax.experimental.pallas import tpu as pltpu
from jax.experimental.pallas import tpu_sc as plsc

B = 16384
D = 128
K = 16
GAMMA = 12.0
BETA = 0.5
G1 = 1.0
G2 = 1.0
EPS = 1e-12

NCORE = 2
NSUB = 16
NW = NCORE * NSUB
SPT = B // NW
GRP = 8
CHUNKS = SPT // GRP
L = 16
CPD = D // L


def _vsqrt(x):
    i = plsc.bitcast(x, jnp.int32)
    g = plsc.bitcast((i >> 1) + jnp.int32(0x1FBD1DF5), jnp.float32)
    for _ in range(3):
        g = 0.5 * (g + x / g)
    return g


def _sc_body(st_hbm, s1_hbm, neg_hbm, e_hbm, r_hbm, t_hbm, c_hbm,
             p_hbm, h_hbm, pa_hbm, out_hbm, part_hbm,
             stm, s1m, negm, idbuf,
             etbuf, txbuf, rbuf, cpbuf, ppbuf, nebuf, ntbuf,
             score_v, part_v, sem_id0, sem_id1, sem_row0, sem_row1):
    wid = lax.axis_index("s") * NCORE + lax.axis_index("c")
    pltpu.sync_copy(st_hbm.at[wid], stm)
    pltpu.sync_copy(s1_hbm.at[wid], s1m)
    pltpu.sync_copy(neg_hbm.at[wid], negm)

    id_sems = (sem_id0, sem_id1)
    row_sems = (sem_row0, sem_row1)

    lane = lax.iota(jnp.int32, 16)
    wv = jnp.where(lane < 2, jnp.float32(G1),
                   jnp.where(lane < 4, jnp.float32(BETA),
                             jnp.where(lane < 6, jnp.float32(G2),
                                       jnp.where(lane == 6, jnp.float32(1.0),
                                                 jnp.float32(0.0)))))

    def issue_ids(j, p):
        ist = stm.at[pl.ds(j * 2 * GRP, 2 * GRP)]
        pltpu.async_copy(h_hbm.at[ist], idbuf.at[p, pl.ds(0, 2 * GRP)],
                         id_sems[p])
        pltpu.async_copy(pa_hbm.at[ist], idbuf.at[p, pl.ds(2 * GRP, 2 * GRP)],
                         id_sems[p])

    def wait_ids(p):
        ist = stm.at[pl.ds(0, 2 * GRP)]
        for off in (0, 2 * GRP):
            pltpu.make_async_copy(h_hbm.at[ist],
                                  idbuf.at[p, pl.ds(off, 2 * GRP)],
                                  id_sems[p]).wait()

    def issue_rows(j, p):
        ist = stm.at[pl.ds(j * 2 * GRP, 2 * GRP)]
        i1 = s1m.at[pl.ds(j * GRP, GRP)]
        ineg = negm.at[pl.ds(j * GRP * K, GRP * K)]
        sem = row_sems[p]
        pltpu.async_copy(e_hbm.at[ist], etbuf.at[p], sem)
        pltpu.async_copy(t_hbm.at[ist], txbuf.at[p], sem)
        pltpu.async_copy(r_hbm.at[i1], rbuf.at[p], sem)
        pltpu.async_copy(c_hbm.at[idbuf.at[p, pl.ds(0, 2 * GRP)]],
                         cpbuf.at[p], sem)
        pltpu.async_copy(p_hbm.at[idbuf.at[p, pl.ds(2 * GRP, 2 * GRP)]],
                         ppbuf.at[p], sem)
        pltpu.async_copy(e_hbm.at[ineg], nebuf.at[p], sem)
        pltpu.async_copy(t_hbm.at[ineg], ntbuf.at[p], sem)

    def wait_rows(p):
        ist = stm.at[pl.ds(0, 2 * GRP)]
        i1 = s1m.at[pl.ds(0, GRP)]
        ineg = negm.at[pl.ds(0, GRP * K)]
        sem = row_sems[p]
        for dst in (etbuf, txbuf, cpbuf, ppbuf):
            pltpu.make_async_copy(e_hbm.at[ist], dst.at[p], sem).wait()
        pltpu.make_async_copy(r_hbm.at[i1], rbuf.at[p], sem).wait()
        for dst in (nebuf, ntbuf):
            pltpu.make_async_copy(e_hbm.at[ineg], dst.at[p], sem).wait()

    def compute(g, p, negacc_in):
        def sample_body(i, carry):
            scorevec, negacc = carry
            z = jnp.zeros((L,), jnp.float32)
            acc_l1 = z
            acc_ht = z
            acc_tt = z
            acc_hc = z
            acc_tc = z
            acc_hp = z
            acc_tp = z
            acc_tr = z
            preds = []
            for c in range(CPD):
                sl = pl.ds(c * L, L)
                h = etbuf[p, i, sl]
                t = etbuf[p, GRP + i, sl]
                ht = txbuf[p, i, sl]
                tt = txbuf[p, GRP + i, sl]
                r = rbuf[p, i, sl]
                hc = cpbuf[p, i, sl]
                tc = cpbuf[p, GRP + i, sl]
                hp = ppbuf[p, i, sl]
                tp = ppbuf[p, GRP + i, sl]
                hcmb = 0.5 * (h + ht)
                tcmb = 0.5 * (t + tt)
                pred = hcmb + r
                u = pred - tcmb
                acc_l1 = acc_l1 + jnp.abs(u)
                dht = h - ht
                acc_ht = acc_ht + dht * dht
                dtt = t - tt
                acc_tt = acc_tt + dtt * dtt
                dhc = hcmb - hc
                acc_hc = acc_hc + dhc * dhc
                dtc = tcmb - tc
                acc_tc = acc_tc + dtc * dtc
                dhp = hc - hp
                acc_hp = acc_hp + dhp * dhp
                dtp = tc - tp
                acc_tp = acc_tp + dtp * dtp
                dtr = u - tt
                acc_tr = acc_tr + dtr * dtr
                preds.append(pred)

            qn = z
            for k in range(K):
                nacc = z
                for c in range(CPD):
                    sl = pl.ds(c * L, L)
                    ne = nebuf[p, i * K + k, sl]
                    nt = ntbuf[p, i * K + k, sl]
                    dn = preds[c] - ne - nt
                    nacc = nacc + dn * dn
                qn = jnp.where(lane == k, jnp.sum(nacc), qn)

            qd = jnp.full((L,), 1.0, jnp.float32)
            qd = jnp.where(lane == 0, 0.25 * jnp.sum(acc_ht), qd)
            qd = jnp.where(lane == 1, 0.25 * jnp.sum(acc_tt), qd)
            qd = jnp.where(lane == 2, jnp.sum(acc_hc), qd)
            qd = jnp.where(lane == 3, jnp.sum(acc_tc), qd)
            qd = jnp.where(lane == 4, jnp.sum(acc_hp), qd)
            qd = jnp.where(lane == 5, jnp.sum(acc_tp), qd)
            qd = jnp.where(lane == 6, jnp.sum(acc_tr), qd)
            sd = _vsqrt(qd + jnp.float32(EPS))
            sn = _vsqrt(qn + jnp.float32(EPS))
            penal = jnp.sum(wv * sd)
            score = jnp.float32(GAMMA) - jnp.sum(acc_l1) - penal
            return (jnp.where(lane == i, score, scorevec), negacc + sn)

        scorevec, negacc = lax.fori_loop(
            0, GRP, sample_body,
            (jnp.zeros((L,), jnp.float32), negacc_in))
        score_v[...] = scorevec
        pltpu.sync_copy(score_v.at[pl.ds(0, GRP)],
                        out_hbm.at[wid, pl.ds(g * GRP, GRP)])
        return negacc

    issue_ids(0, 0)
    issue_ids(1, 1)
    wait_ids(0)
    issue_rows(0, 0)

    def pair_body(tt, negacc):
        g0 = 2 * tt
        more = tt < CHUNKS // 2 - 1

        wait_ids(1)
        issue_rows(g0 + 1, 1)
        wait_rows(0)

        @pl.when(more)
        def _():
            issue_ids(g0 + 2, 0)

        negacc = compute(g0, 0, negacc)

        @pl.when(more)
        def _():
            wait_ids(0)
            issue_rows(g0 + 2, 0)

        wait_rows(1)

        @pl.when(more)
        def _():
            issue_ids(g0 + 3, 1)

        negacc = compute(g0 + 1, 1, negacc)
        return negacc

    negacc = lax.fori_loop(0, CHUNKS // 2, pair_body,
                           jnp.zeros((L,), jnp.float32))

    for c in range(CPD):
        part_v[pl.ds(c * L, L)] = jnp.zeros((L,), jnp.float32)
    part_v[pl.ds(0, L)] = negacc
    pltpu.sync_copy(part_v, part_hbm.at[wid])


_sc_kernel = functools.partial(
    pl.kernel,
    out_type=(jax.ShapeDtypeStruct((NW, SPT), jnp.float32),
              jax.ShapeDtypeStruct((NW, D), jnp.float32)),
    mesh=plsc.VectorSubcoreMesh(core_axis_name="c", subcore_axis_name="s"),
    compiler_params=pltpu.CompilerParams(needs_layout_passes=False),
    scratch_types=[
        pltpu.VMEM((SPT * 2,), jnp.int32),
        pltpu.VMEM((SPT,), jnp.int32),
        pltpu.VMEM((SPT * K,), jnp.int32),
        pltpu.VMEM((2, 4 * GRP), jnp.int32),
        pltpu.VMEM((2, 2 * GRP, D), jnp.float32),
        pltpu.VMEM((2, 2 * GRP, D), jnp.float32),
        pltpu.VMEM((2, GRP, D), jnp.float32),
        pltpu.VMEM((2, 2 * GRP, D), jnp.float32),
        pltpu.VMEM((2, 2 * GRP, D), jnp.float32),
        pltpu.VMEM((2, GRP * K, D), jnp.float32),
        pltpu.VMEM((2, GRP * K, D), jnp.float32),
        pltpu.VMEM((L,), jnp.float32),
        pltpu.VMEM((D,), jnp.float32),
        pltpu.SemaphoreType.DMA,
        pltpu.SemaphoreType.DMA,
        pltpu.SemaphoreType.DMA,
        pltpu.SemaphoreType.DMA,
    ],
)


def _run_sc(*args):
    return _sc_kernel(_sc_body)(*args)


def _combine_body(base_ref, part_ref, out_ref):
    negmean = jnp.sum(part_ref[...]) * jnp.float32(1.0 / (B * K))
    out_ref[...] = base_ref[...] + negmean


_combine = pl.pallas_call(
    _combine_body,
    out_shape=jax.ShapeDtypeStruct((NW, SPT), jnp.float32),
)


def kernel(sample, entity_embedding, relation_embedding, entity_text_embeddings,
           cluster_embedding, parent_cluster_embedding, entity_hierarchy,
           entity_parent, neg_idx):
    s0 = sample[:, 0].reshape(NW, CHUNKS, GRP)
    s2 = sample[:, 2].reshape(NW, CHUNKS, GRP)
    st = jnp.concatenate([s0, s2], axis=2).reshape(NW, SPT * 2)
    s1 = sample[:, 1].reshape(NW, SPT)
    neg2 = neg_idx.reshape(NW, SPT * K)
    base, parts = _run_sc(st, s1, neg2, entity_embedding,
                          relation_embedding, entity_text_embeddings,
                          cluster_embedding, parent_cluster_embedding,
                          entity_hierarchy, entity_parent)
    out = _combine(base, parts)
    return out.reshape(B, 1)


# speedup vs baseline: 2.0106x; 1.0101x over previous
import functools

import jax
import jax.numpy as jnp
from jax import lax
from jax.experimental import pallas as pl
from jax.experimental.pallas import tpu as pltpu
from jax.experimental.pallas import tpu_sc as plsc

B = 16384
D = 128
K = 16
GAMMA = 12.0
BETA = 0.5
G1 = 1.0
G2 = 1.0
EPS = 1e-12

NCORE = 2
NSUB = 16
NW = NCORE * NSUB
SPT = B // NW
GRP = 8
CHUNKS = SPT // GRP
L = 16
CPD = D // L


def _vsqrt(x):
    i = plsc.bitcast(x, jnp.int32)
    g = plsc.bitcast((i >> 1) + jnp.int32(0x1FBD1DF5), jnp.float32)
    for _ in range(3):
        g = 0.5 * (g + x / g)
    return g


def _sc_body(st_hbm, s1_hbm, neg_hbm, e_hbm, r_hbm, t_hbm, c_hbm,
             p_hbm, h_hbm, pa_hbm, out_hbm, part_hbm,
             stm, s1m, negm, idbuf,
             etbuf, txbuf, rbuf, cpbuf, ppbuf, nebuf, ntbuf,
             score_v, part_v, sem_id0, sem_id1, sem_row0, sem_row1):
    wid = lax.axis_index("s") * NCORE + lax.axis_index("c")
    pltpu.sync_copy(st_hbm.at[wid], stm)
    pltpu.sync_copy(s1_hbm.at[wid], s1m)
    pltpu.sync_copy(neg_hbm.at[wid], negm)

    id_sems = (sem_id0, sem_id1)
    row_sems = (sem_row0, sem_row1)

    lane = lax.iota(jnp.int32, 16)
    wv = jnp.where(lane < 2, jnp.float32(G1),
                   jnp.where(lane < 4, jnp.float32(BETA),
                             jnp.where(lane < 6, jnp.float32(G2),
                                       jnp.where(lane == 6, jnp.float32(1.0),
                                                 jnp.float32(0.0)))))

    def issue_ids(j, p):
        ist = stm.at[pl.ds(j * 2 * GRP, 2 * GRP)]
        pltpu.async_copy(h_hbm.at[ist], idbuf.at[p, pl.ds(0, 2 * GRP)],
                         id_sems[p])
        pltpu.async_copy(pa_hbm.at[ist], idbuf.at[p, pl.ds(2 * GRP, 2 * GRP)],
                         id_sems[p])

    def wait_ids(p):
        ist = stm.at[pl.ds(0, 2 * GRP)]
        for off in (0, 2 * GRP):
            pltpu.make_async_copy(h_hbm.at[ist],
                                  idbuf.at[p, pl.ds(off, 2 * GRP)],
                                  id_sems[p]).wait()

    def issue_rows(j, p):
        ist = stm.at[pl.ds(j * 2 * GRP, 2 * GRP)]
        i1 = s1m.at[pl.ds(j * GRP, GRP)]
        ineg = negm.at[pl.ds(j * GRP * K, GRP * K)]
        sem = row_sems[p]
        pltpu.async_copy(e_hbm.at[ist], etbuf.at[p], sem)
        pltpu.async_copy(t_hbm.at[ist], txbuf.at[p], sem)
        pltpu.async_copy(r_hbm.at[i1], rbuf.at[p], sem)
        pltpu.async_copy(c_hbm.at[idbuf.at[p, pl.ds(0, 2 * GRP)]],
                         cpbuf.at[p], sem)
        pltpu.async_copy(p_hbm.at[idbuf.at[p, pl.ds(2 * GRP, 2 * GRP)]],
                         ppbuf.at[p], sem)
        pltpu.async_copy(e_hbm.at[ineg], nebuf.at[p], sem)
        pltpu.async_copy(t_hbm.at[ineg], ntbuf.at[p], sem)

    def wait_rows(p):
        ist = stm.at[pl.ds(0, 2 * GRP)]
        i1 = s1m.at[pl.ds(0, GRP)]
        ineg = negm.at[pl.ds(0, GRP * K)]
        sem = row_sems[p]
        for dst in (etbuf, txbuf, cpbuf, ppbuf):
            pltpu.make_async_copy(e_hbm.at[ist], dst.at[p], sem).wait()
        pltpu.make_async_copy(r_hbm.at[i1], rbuf.at[p], sem).wait()
        for dst in (nebuf, ntbuf):
            pltpu.make_async_copy(e_hbm.at[ineg], dst.at[p], sem).wait()

    def compute(g, p, negacc_in):
        def sample_body(i, carry):
            scorevec, negacc = carry
            z = jnp.zeros((L,), jnp.float32)
            acc_l1 = z
            acc_ht = z
            acc_tt = z
            acc_hc = z
            acc_tc = z
            acc_hp = z
            acc_tp = z
            acc_tr = z
            preds = []
            for c in range(CPD):
                sl = pl.ds(c * L, L)
                h = etbuf[p, i, sl]
                t = etbuf[p, GRP + i, sl]
                ht = txbuf[p, i, sl]
                tt = txbuf[p, GRP + i, sl]
                r = rbuf[p, i, sl]
                hc = cpbuf[p, i, sl]
                tc = cpbuf[p, GRP + i, sl]
                hp = ppbuf[p, i, sl]
                tp = ppbuf[p, GRP + i, sl]
                hcmb = 0.5 * (h + ht)
                tcmb = 0.5 * (t + tt)
                pred = hcmb + r
                u = pred - tcmb
                acc_l1 = acc_l1 + jnp.abs(u)
                dht = h - ht
                acc_ht = acc_ht + dht * dht
                dtt = t - tt
                acc_tt = acc_tt + dtt * dtt
                dhc = hcmb - hc
                acc_hc = acc_hc + dhc * dhc
                dtc = tcmb - tc
                acc_tc = acc_tc + dtc * dtc
                dhp = hc - hp
                acc_hp = acc_hp + dhp * dhp
                dtp = tc - tp
                acc_tp = acc_tp + dtp * dtp
                dtr = u - tt
                acc_tr = acc_tr + dtr * dtr
                preds.append(pred)

            qd = jnp.full((L,), 1.0, jnp.float32)
            qd = jnp.where(lane == 0, 0.25 * jnp.sum(acc_ht), qd)
            qd = jnp.where(lane == 1, 0.25 * jnp.sum(acc_tt), qd)
            qd = jnp.where(lane == 2, jnp.sum(acc_hc), qd)
            qd = jnp.where(lane == 3, jnp.sum(acc_tc), qd)
            qd = jnp.where(lane == 4, jnp.sum(acc_hp), qd)
            qd = jnp.where(lane == 5, jnp.sum(acc_tp), qd)
            qd = jnp.where(lane == 6, jnp.sum(acc_tr), qd)
            sl1 = jnp.sum(acc_l1)

            qn = z
            for k in range(K):
                nacc = z
                for c in range(CPD):
                    sl = pl.ds(c * L, L)
                    ne = nebuf[p, i * K + k, sl]
                    nt = ntbuf[p, i * K + k, sl]
                    dn = preds[c] - ne - nt
                    nacc = nacc + dn * dn
                qn = jnp.where(lane == k, jnp.sum(nacc), qn)

            sd = _vsqrt(qd + jnp.float32(EPS))
            sn = _vsqrt(qn + jnp.float32(EPS))
            penal = jnp.sum(wv * sd)
            score = jnp.float32(GAMMA) - sl1 - penal
            return (jnp.where(lane == i, score, scorevec), negacc + sn)

        scorevec, negacc = lax.fori_loop(
            0, GRP, sample_body,
            (jnp.zeros((L,), jnp.float32), negacc_in))
        score_v[...] = scorevec
        pltpu.sync_copy(score_v.at[pl.ds(0, GRP)],
                        out_hbm.at[wid, pl.ds(g * GRP, GRP)])
        return negacc

    issue_ids(0, 0)
    issue_ids(1, 1)
    wait_ids(0)
    issue_rows(0, 0)

    def pair_body(tt, negacc):
        g0 = 2 * tt
        more = tt < CHUNKS // 2 - 1

        wait_ids(1)
        issue_rows(g0 + 1, 1)
        wait_rows(0)

        @pl.when(more)
        def _():
            issue_ids(g0 + 2, 0)

        negacc = compute(g0, 0, negacc)

        @pl.when(more)
        def _():
            wait_ids(0)
            issue_rows(g0 + 2, 0)

        wait_rows(1)

        @pl.when(more)
        def _():
            issue_ids(g0 + 3, 1)

        negacc = compute(g0 + 1, 1, negacc)
        return negacc

    negacc = lax.fori_loop(0, CHUNKS // 2, pair_body,
                           jnp.zeros((L,), jnp.float32))

    for c in range(CPD):
        part_v[pl.ds(c * L, L)] = jnp.zeros((L,), jnp.float32)
    part_v[pl.ds(0, L)] = negacc
    pltpu.sync_copy(part_v, part_hbm.at[wid])


_sc_kernel = functools.partial(
    pl.kernel,
    out_type=(jax.ShapeDtypeStruct((NW, SPT), jnp.float32),
              jax.ShapeDtypeStruct((NW, D), jnp.float32)),
    mesh=plsc.VectorSubcoreMesh(core_axis_name="c", subcore_axis_name="s"),
    compiler_params=pltpu.CompilerParams(needs_layout_passes=False),
    scratch_types=[
        pltpu.VMEM((SPT * 2,), jnp.int32),
        pltpu.VMEM((SPT,), jnp.int32),
        pltpu.VMEM((SPT * K,), jnp.int32),
        pltpu.VMEM((2, 4 * GRP), jnp.int32),
        pltpu.VMEM((2, 2 * GRP, D), jnp.float32),
        pltpu.VMEM((2, 2 * GRP, D), jnp.float32),
        pltpu.VMEM((2, GRP, D), jnp.float32),
        pltpu.VMEM((2, 2 * GRP, D), jnp.float32),
        pltpu.VMEM((2, 2 * GRP, D), jnp.float32),
        pltpu.VMEM((2, GRP * K, D), jnp.float32),
        pltpu.VMEM((2, GRP * K, D), jnp.float32),
        pltpu.VMEM((L,), jnp.float32),
        pltpu.VMEM((D,), jnp.float32),
        pltpu.SemaphoreType.DMA,
        pltpu.SemaphoreType.DMA,
        pltpu.SemaphoreType.DMA,
        pltpu.SemaphoreType.DMA,
    ],
)


def _run_sc(*args):
    return _sc_kernel(_sc_body)(*args)


def _combine_body(base_ref, part_ref, out_ref):
    negmean = jnp.sum(part_ref[...]) * jnp.float32(1.0 / (B * K))
    out_ref[...] = base_ref[...] + negmean


_combine = pl.pallas_call(
    _combine_body,
    out_shape=jax.ShapeDtypeStruct((NW, SPT), jnp.float32),
)


def kernel(sample, entity_embedding, relation_embedding, entity_text_embeddings,
           cluster_embedding, parent_cluster_embedding, entity_hierarchy,
           entity_parent, neg_idx):
    s0 = sample[:, 0].reshape(NW, CHUNKS, GRP)
    s2 = sample[:, 2].reshape(NW, CHUNKS, GRP)
    st = jnp.concatenate([s0, s2], axis=2).reshape(NW, SPT * 2)
    s1 = sample[:, 1].reshape(NW, SPT)
    neg2 = neg_idx.reshape(NW, SPT * K)
    base, parts = _run_sc(st, s1, neg2, entity_embedding,
                          relation_embedding, entity_text_embeddings,
                          cluster_embedding, parent_cluster_embedding,
                          entity_hierarchy, entity_parent)
    out = _combine(base, parts)
    return out.reshape(B, 1)
